# Initial kernel scaffold; baseline (speedup 1.0000x reference)
import jax
import jax.numpy as jnp
from jax.experimental import pallas as pl


def kernel(x, edge_index, edge_attr, batch, nn1_w1, nn1_b1, nn1_w2, nn1_b2, root1, bias1, nn2_w1, nn2_b1, nn2_w2, nn2_b2, root2, bias2, nn3_w1, nn3_b1, nn3_w2, nn3_b2, root3, bias3, lstm_wih, lstm_whh, lstm_bih, lstm_bhh, lin1_w, lin1_b, lin2_w, lin2_b, linf_w, linf_b):
    raise NotImplementedError("write your pallas kernel here")


# speedup vs baseline: 1.1071x; 1.1071x over previous
import functools

import jax
import jax.numpy as jnp
from jax import lax
from jax.experimental import pallas as pl
from jax.experimental.pallas import tpu as pltpu
from jax.experimental.pallas import tpu_sc as plsc

N = 10000
E = 160000
B = 64

NC = 2
NS = 16
NW = NC * NS
CH = 128
NCHUNK = E // CH
HALF = N // NC
ACC_ROWS = HALF + 8
TS = ACC_ROWS // NS
EB = 2000

_mesh = functools.partial(
    plsc.VectorSubcoreMesh, core_axis_name="c", subcore_axis_name="s",
    num_cores=NC, num_subcores=NS)


def _make_gather(width):
  npw = -(-NCHUNK // NW)

  @functools.partial(
      pl.kernel,
      out_type=jax.ShapeDtypeStruct((E, width), jnp.float32),
      mesh=_mesh(),
      scratch_types=[
          pltpu.VMEM((CH,), jnp.int32),
          pltpu.VMEM((CH, width), jnp.float32),
          pltpu.SemaphoreType.DMA,
      ],
      compiler_params=pltpu.CompilerParams(use_tc_tiling_on_sc=False),
  )
  def gather(x_hbm, src_hbm, xs_hbm, idx_v, rows_v, sem):
    wid = lax.axis_index("s") * NC + lax.axis_index("c")

    def body(j, carry):
      c = wid + NW * j

      @pl.when(c < NCHUNK)
      def _():
        base = c * CH
        pltpu.sync_copy(src_hbm.at[pl.ds(base, CH)], idx_v)
        pltpu.async_copy(x_hbm.at[idx_v], rows_v, sem).wait()
        pltpu.sync_copy(rows_v, xs_hbm.at[pl.ds(base, CH)])

      return carry

    lax.fori_loop(0, npw, body, 0)

  return gather


def _make_scatter(width):
  npt = -(-NCHUNK // NS)

  @functools.partial(
      pl.kernel,
      out_type=jax.ShapeDtypeStruct((NC, ACC_ROWS, width), jnp.float32),
      mesh=_mesh(),
      scratch_types=[
          pltpu.VMEM((CH,), jnp.int32),
          pltpu.VMEM((1, CH), jnp.int32),
          pltpu.VMEM((CH, width), jnp.float32),
          pltpu.VMEM_SHARED((ACC_ROWS, width), jnp.float32),
      ],
      compiler_params=pltpu.CompilerParams(use_tc_tiling_on_sc=False),
  )
  def scatter(msg_hbm, dst_hbm, zero_hbm, out_hbm, dbuf, ibuf, mbuf, acc_sh):
    core = lax.axis_index("c")
    sid = lax.axis_index("s")
    base_node = core * HALF

    pltpu.sync_copy(zero_hbm, acc_sh.at[pl.ds(sid * TS, TS)])
    plsc.subcore_barrier()

    def body(j, carry):
      c = sid + NS * j

      @pl.when(c < NCHUNK)
      def _():
        eb = c * CH
        pltpu.sync_copy(dst_hbm.at[pl.ds(eb, CH)], dbuf)
        for k in range(CH // 16):
          v = dbuf[pl.ds(k * 16, 16)]
          rel = v - base_node
          ok = (rel >= 0) & (rel < HALF)
          ibuf[0, pl.ds(k * 16, 16)] = jnp.where(ok, rel, HALF)
        pltpu.sync_copy(msg_hbm.at[pl.ds(eb, CH)], mbuf)
        pltpu.sync_copy(mbuf, acc_sh.at[ibuf.at[0]], add=True)

      return carry

    lax.fori_loop(0, npt, body, 0)
    plsc.subcore_barrier()
    pltpu.sync_copy(acc_sh.at[pl.ds(sid * TS, TS)],
                    out_hbm.at[core, pl.ds(sid * TS, TS)])

  return scatter


def _make_dense(cin, cout, kw, wout, ones_col, xs_w):
  def body(ea_ref, xs_ref, w1_ref, b1_ref, w2_ref, b2_ref, o_ref):
    h = jnp.maximum(ea_ref[...] @ w1_ref[...] + b1_ref[...], 0.0)
    w = h @ w2_ref[...] + b2_ref[...]
    xs = xs_ref[...]
    msg = xs[:, 0:1] * w[:, 0:cout]
    for i in range(1, cin):
      msg = msg + xs[:, i:i + 1] * w[:, i * cout:(i + 1) * cout]
    if wout > cout:
      pads = [msg]
      if ones_col:
        pads.append(jnp.ones((EB, 1), jnp.float32))
        pads.append(jnp.zeros((EB, wout - cout - 1), jnp.float32))
      else:
        pads.append(jnp.zeros((EB, wout - cout), jnp.float32))
      msg = jnp.concatenate(pads, axis=-1)
    o_ref[...] = msg

  return pl.pallas_call(
      body,
      grid=(E // EB,),
      in_specs=[
          pl.BlockSpec((EB, 4), lambda i: (i, 0)),
          pl.BlockSpec((EB, xs_w), lambda i: (i, 0)),
          pl.BlockSpec((4, 32), lambda i: (0, 0)),
          pl.BlockSpec((1, 32), lambda i: (0, 0)),
          pl.BlockSpec((32, kw), lambda i: (0, 0)),
          pl.BlockSpec((1, kw), lambda i: (0, 0)),
      ],
      out_specs=pl.BlockSpec((EB, wout), lambda i: (i, 0)),
      out_shape=jax.ShapeDtypeStruct((E, wout), jnp.float32),
  )


def _root_body(x_ref, w_ref, b_ref, o_ref):
  o_ref[...] = x_ref[...] @ w_ref[...] + b_ref[...]


def _root_call(cout):
  return pl.pallas_call(
      _root_body, out_shape=jax.ShapeDtypeStruct((N, cout), jnp.float32))


def _finish1_body(acc_ref, r1_ref, w_ref, b_ref, x1_ref, invc_ref, r2_ref):
  a = jnp.concatenate([acc_ref[0, :HALF, :], acc_ref[1, :HALF, :]], axis=0)
  cnt = a[:, 24:25]
  invc = 1.0 / jnp.maximum(cnt, 1.0)
  y = jnp.maximum(a[:, :24] * invc + r1_ref[...], 0.0)
  x1_ref[...] = jnp.concatenate([y, jnp.zeros((N, 8), jnp.float32)], axis=-1)
  invc_ref[...] = invc
  r2_ref[...] = y @ w_ref[...] + b_ref[...]


_finish1 = pl.pallas_call(
    _finish1_body,
    out_shape=(
        jax.ShapeDtypeStruct((N, 32), jnp.float32),
        jax.ShapeDtypeStruct((N, 1), jnp.float32),
        jax.ShapeDtypeStruct((N, 16), jnp.float32),
    ),
)


def _finish2_body(acc_ref, invc_ref, r2_ref, w_ref, b_ref, x2_ref, r3_ref):
  a = jnp.concatenate([acc_ref[0, :HALF, :], acc_ref[1, :HALF, :]], axis=0)
  y = jnp.maximum(a * invc_ref[...] + r2_ref[...], 0.0)
  x2_ref[...] = y
  r3_ref[...] = y @ w_ref[...] + b_ref[...]


_finish2 = pl.pallas_call(
    _finish2_body,
    out_shape=(
        jax.ShapeDtypeStruct((N, 16), jnp.float32),
        jax.ShapeDtypeStruct((N, 8), jnp.float32),
    ),
)


def _s2s_body(acc_ref, invc_ref, r3_ref, bt_ref, wih_ref, whh_ref, lb_ref,
              l1w_ref, l1b_ref, l2w_ref, l2b_ref, lfw_ref, lfb_ref, o_ref):
  a3 = jnp.concatenate([acc_ref[0, :HALF, :], acc_ref[1, :HALF, :]], axis=0)
  x3 = jnp.maximum(a3[:, :8] * invc_ref[...] + r3_ref[...], 0.0)
  bt = bt_ref[...]
  cols = lax.broadcasted_iota(jnp.int32, (N, B), 1)
  oh = jnp.where(bt == cols, 1.0, 0.0)

  q_star = jnp.zeros((B, 16), jnp.float32)
  h = jnp.zeros((B, 8), jnp.float32)
  c = jnp.zeros((B, 8), jnp.float32)
  for _ in range(2):
    gates = q_star @ wih_ref[...] + h @ whh_ref[...] + lb_ref[...]
    ig = jax.nn.sigmoid(gates[:, 0:8])
    fg = jax.nn.sigmoid(gates[:, 8:16])
    gg = jnp.tanh(gates[:, 16:24])
    og = jax.nn.sigmoid(gates[:, 24:32])
    c = fg * c + ig * gg
    h = og * jnp.tanh(c)
    q = h
    qb = oh @ q
    e = jnp.sum(x3 * qb, axis=1, keepdims=True)
    masked = jnp.where(oh > 0.0, e, -jnp.inf)
    emax = jnp.max(masked, axis=0, keepdims=True)
    emax = jnp.where(jnp.isfinite(emax), emax, 0.0)
    emaxb = jnp.sum(oh * emax, axis=1, keepdims=True)
    aa = jnp.exp(e - emaxb)
    denom = jnp.sum(oh * aa, axis=0, keepdims=True)
    denomb = jnp.sum(oh * denom, axis=1, keepdims=True)
    aa = aa / (denomb + 1e-16)
    r = lax.dot_general(oh, aa * x3, (((0,), (0,)), ((), ())))
    q_star = jnp.concatenate([q, r], axis=-1)

  y = jnp.maximum(q_star @ l1w_ref[...] + l1b_ref[...], 0.0)
  y = jnp.maximum(y @ l2w_ref[...] + l2b_ref[...], 0.0)
  o_ref[...] = y @ lfw_ref[...] + lfb_ref[...]


_s2s = pl.pallas_call(
    _s2s_body, out_shape=jax.ShapeDtypeStruct((B, 1), jnp.float32))


_gather16 = _make_gather(16)
_gather32 = _make_gather(32)
_scatter32 = _make_scatter(32)
_scatter16 = _make_scatter(16)
_dense1 = _make_dense(16, 24, 384, 32, True, 16)
_dense2 = _make_dense(24, 16, 384, 16, False, 32)
_dense3 = _make_dense(16, 8, 128, 16, False, 16)


def kernel(x, edge_index, edge_attr, batch, nn1_w1, nn1_b1, nn1_w2, nn1_b2,
           root1, bias1, nn2_w1, nn2_b1, nn2_w2, nn2_b2, root2, bias2,
           nn3_w1, nn3_b1, nn3_w2, nn3_b2, root3, bias3, lstm_wih, lstm_whh,
           lstm_bih, lstm_bhh, lin1_w, lin1_b, lin2_w, lin2_b, linf_w,
           linf_b):
  src = edge_index[0]
  dst = edge_index[1]
  zero32 = jnp.zeros((TS, 32), jnp.float32)
  zero16 = jnp.zeros((TS, 16), jnp.float32)
  row = lambda v: v.reshape(1, -1)

  r1 = _root_call(24)(x, root1, row(bias1))
  xs1 = _gather16(x, src)
  msg1 = _dense1(edge_attr, xs1, nn1_w1, row(nn1_b1), nn1_w2, row(nn1_b2))
  acc1 = _scatter32(msg1, dst, zero32)
  x1p, invc, r2 = _finish1(acc1, r1, root2, row(bias2))

  xs2 = _gather32(x1p, src)
  msg2 = _dense2(edge_attr, xs2, nn2_w1, row(nn2_b1), nn2_w2, row(nn2_b2))
  acc2 = _scatter16(msg2, dst, zero16)
  x2, r3 = _finish2(acc2, invc, r2, root3, row(bias3))

  xs3 = _gather16(x2, src)
  msg3 = _dense3(edge_attr, xs3, nn3_w1, row(nn3_b1), nn3_w2, row(nn3_b2))
  acc3 = _scatter16(msg3, dst, zero16)

  out = _s2s(acc3, invc, r3, batch.reshape(N, 1), lstm_wih, lstm_whh,
             row(lstm_bih + lstm_bhh), lin1_w, row(lin1_b), lin2_w,
             row(lin2_b), linf_w, row(linf_b))
  return out.reshape(B)


# speedup vs baseline: 2.5599x; 2.3123x over previous
import functools

import jax
import jax.numpy as jnp
from jax import lax
from jax.experimental import pallas as pl
from jax.experimental.pallas import tpu as pltpu
from jax.experimental.pallas import tpu_sc as plsc

N = 10000
E = 160000
B = 64

NC = 2
NS = 16
NW = NC * NS
CH = 128
NCHUNK = E // CH
HALF = N // NC
ACC_ROWS = HALF + 8
TS = ACC_ROWS // NS
EB = 2000

_mesh = functools.partial(
    plsc.VectorSubcoreMesh, core_axis_name="c", subcore_axis_name="s",
    num_cores=NC, num_subcores=NS)


def _make_gather(width):
  npw = -(-NCHUNK // NW)

  @functools.partial(
      pl.kernel,
      out_type=jax.ShapeDtypeStruct((E, width), jnp.float32),
      mesh=_mesh(),
      scratch_types=[
          pltpu.VMEM((CH,), jnp.int32),
          pltpu.VMEM((CH, width), jnp.float32),
          pltpu.SemaphoreType.DMA,
      ],
      compiler_params=pltpu.CompilerParams(use_tc_tiling_on_sc=False),
  )
  def gather(x_hbm, src_hbm, xs_hbm, idx_v, rows_v, sem):
    wid = lax.axis_index("s") * NC + lax.axis_index("c")

    def body(j, carry):
      c = wid + NW * j

      @pl.when(c < NCHUNK)
      def _():
        base = c * CH
        pltpu.sync_copy(src_hbm.at[pl.ds(base, CH)], idx_v)
        pltpu.async_copy(x_hbm.at[idx_v], rows_v, sem).wait()
        pltpu.sync_copy(rows_v, xs_hbm.at[pl.ds(base, CH)])

      return carry

    lax.fori_loop(0, npw, body, 0)

  return gather


def _make_scatter(width):
  npt = -(-NCHUNK // NS)

  @functools.partial(
      pl.kernel,
      out_type=jax.ShapeDtypeStruct((NC, ACC_ROWS, width), jnp.float32),
      mesh=_mesh(),
      scratch_types=[
          pltpu.VMEM((CH,), jnp.int32),
          pltpu.VMEM((1, CH), jnp.int32),
          pltpu.VMEM((CH, width), jnp.float32),
          pltpu.VMEM_SHARED((ACC_ROWS, width), jnp.float32),
      ],
      compiler_params=pltpu.CompilerParams(use_tc_tiling_on_sc=False),
  )
  def scatter(msg_hbm, dst_hbm, zero_hbm, out_hbm, dbuf, ibuf, mbuf, acc_sh):
    core = lax.axis_index("c")
    sid = lax.axis_index("s")
    base_node = core * HALF

    pltpu.sync_copy(zero_hbm, acc_sh.at[pl.ds(sid * TS, TS)])
    plsc.subcore_barrier()

    def body(j, carry):
      c = sid + NS * j

      @pl.when(c < NCHUNK)
      def _():
        eb = c * CH
        pltpu.sync_copy(dst_hbm.at[pl.ds(eb, CH)], dbuf)
        for k in range(CH // 16):
          v = dbuf[pl.ds(k * 16, 16)]
          rel = v - base_node
          ok = (rel >= 0) & (rel < HALF)
          ibuf[0, pl.ds(k * 16, 16)] = jnp.where(ok, rel, HALF)
        pltpu.sync_copy(msg_hbm.at[pl.ds(eb, CH)], mbuf)
        pltpu.sync_copy(mbuf, acc_sh.at[ibuf.at[0]], add=True)

      return carry

    lax.fori_loop(0, npt, body, 0)
    plsc.subcore_barrier()
    pltpu.sync_copy(acc_sh.at[pl.ds(sid * TS, TS)],
                    out_hbm.at[core, pl.ds(sid * TS, TS)])

  return scatter


def _make_dense(cin, cout, xs_w):
  ku = cin * 32

  def body(ea_ref, xs_ref, w1_ref, b1_ref, rx_ref, rh_ref, w2r_ref, b2r_ref,
           o_ref):
    h = jnp.maximum(ea_ref[...] @ w1_ref[...] + b1_ref[...], 0.0)
    xc = xs_ref[...][:, :cin]
    u = (xc @ rx_ref[...]) * (h @ rh_ref[...])
    o_ref[...] = u @ w2r_ref[...] + xc @ b2r_ref[...]

  return pl.pallas_call(
      body,
      grid=(E // EB,),
      in_specs=[
          pl.BlockSpec((EB, 4), lambda i: (i, 0)),
          pl.BlockSpec((EB, xs_w), lambda i: (i, 0)),
          pl.BlockSpec((4, 32), lambda i: (0, 0)),
          pl.BlockSpec((1, 32), lambda i: (0, 0)),
          pl.BlockSpec((cin, ku), lambda i: (0, 0)),
          pl.BlockSpec((32, ku), lambda i: (0, 0)),
          pl.BlockSpec((ku, cout), lambda i: (0, 0)),
          pl.BlockSpec((cin, cout), lambda i: (0, 0)),
      ],
      out_specs=pl.BlockSpec((EB, cout), lambda i: (i, 0)),
      out_shape=jax.ShapeDtypeStruct((E, cout), jnp.float32),
  )


def _dense_weights(cin, cout, w2, b2):
  rx = jnp.kron(jnp.eye(cin, dtype=jnp.float32),
                jnp.ones((1, 32), jnp.float32))
  rh = jnp.kron(jnp.ones((1, cin), jnp.float32),
                jnp.eye(32, dtype=jnp.float32))
  w2r = w2.reshape(32, cin, cout).transpose(1, 0, 2).reshape(cin * 32, cout)
  b2r = b2.reshape(cin, cout)
  return rx, rh, w2r, b2r


def _root_body(x_ref, w_ref, b_ref, o_ref):
  o_ref[...] = x_ref[...] @ w_ref[...] + b_ref[...]


def _root_call(cout):
  return pl.pallas_call(
      _root_body, out_shape=jax.ShapeDtypeStruct((N, cout), jnp.float32))


def _finish1_body(acc_ref, cacc_ref, r1_ref, w_ref, b_ref, x1_ref, invc_ref,
                  r2_ref):
  a = jnp.concatenate([acc_ref[0, :HALF, :], acc_ref[1, :HALF, :]], axis=0)
  cnt = jnp.concatenate([cacc_ref[0, :HALF, :], cacc_ref[1, :HALF, :]],
                        axis=0)[:, 0:1]
  invc = 1.0 / jnp.maximum(cnt, 1.0)
  y = jnp.maximum(a * invc + r1_ref[...], 0.0)
  x1_ref[...] = jnp.concatenate([y, jnp.zeros((N, 8), jnp.float32)], axis=-1)
  invc_ref[...] = invc
  r2_ref[...] = y @ w_ref[...] + b_ref[...]


_finish1 = pl.pallas_call(
    _finish1_body,
    out_shape=(
        jax.ShapeDtypeStruct((N, 32), jnp.float32),
        jax.ShapeDtypeStruct((N, 1), jnp.float32),
        jax.ShapeDtypeStruct((N, 16), jnp.float32),
    ),
)


def _finish2_body(acc_ref, invc_ref, r2_ref, w_ref, b_ref, x2_ref, r3_ref):
  a = jnp.concatenate([acc_ref[0, :HALF, :], acc_ref[1, :HALF, :]], axis=0)
  y = jnp.maximum(a * invc_ref[...] + r2_ref[...], 0.0)
  x2_ref[...] = y
  r3_ref[...] = y @ w_ref[...] + b_ref[...]


_finish2 = pl.pallas_call(
    _finish2_body,
    out_shape=(
        jax.ShapeDtypeStruct((N, 16), jnp.float32),
        jax.ShapeDtypeStruct((N, 8), jnp.float32),
    ),
)


def _s2s_body(acc_ref, invc_ref, r3_ref, bt_ref, wih_ref, whh_ref, lb_ref,
              l1w_ref, l1b_ref, l2w_ref, l2b_ref, lfw_ref, lfb_ref, o_ref):
  a3 = jnp.concatenate([acc_ref[0, :HALF, :], acc_ref[1, :HALF, :]], axis=0)
  x3 = jnp.maximum(a3 * invc_ref[...] + r3_ref[...], 0.0)
  bt = bt_ref[...]
  cols = lax.broadcasted_iota(jnp.int32, (N, B), 1)
  oh = jnp.where(bt == cols, 1.0, 0.0)

  q_star = jnp.zeros((B, 16), jnp.float32)
  h = jnp.zeros((B, 8), jnp.float32)
  c = jnp.zeros((B, 8), jnp.float32)
  for _ in range(2):
    gates = q_star @ wih_ref[...] + h @ whh_ref[...] + lb_ref[...]
    ig = jax.nn.sigmoid(gates[:, 0:8])
    fg = jax.nn.sigmoid(gates[:, 8:16])
    gg = jnp.tanh(gates[:, 16:24])
    og = jax.nn.sigmoid(gates[:, 24:32])
    c = fg * c + ig * gg
    h = og * jnp.tanh(c)
    q = h
    qb = oh @ q
    e = jnp.sum(x3 * qb, axis=1, keepdims=True)
    masked = jnp.where(oh > 0.0, e, -jnp.inf)
    emax = jnp.max(masked, axis=0, keepdims=True)
    emax = jnp.where(jnp.isfinite(emax), emax, 0.0)
    emaxb = jnp.sum(oh * emax, axis=1, keepdims=True)
    aa = jnp.exp(e - emaxb)
    denom = jnp.sum(oh * aa, axis=0, keepdims=True)
    denomb = jnp.sum(oh * denom, axis=1, keepdims=True)
    aa = aa / (denomb + 1e-16)
    r = lax.dot_general(oh, aa * x3, (((0,), (0,)), ((), ())))
    q_star = jnp.concatenate([q, r], axis=-1)

  y = jnp.maximum(q_star @ l1w_ref[...] + l1b_ref[...], 0.0)
  y = jnp.maximum(y @ l2w_ref[...] + l2b_ref[...], 0.0)
  o_ref[...] = y @ lfw_ref[...] + lfb_ref[...]


_s2s = pl.pallas_call(
    _s2s_body, out_shape=jax.ShapeDtypeStruct((B, 1), jnp.float32))


_gather16 = _make_gather(16)
_gather32 = _make_gather(32)
_scatter24 = _make_scatter(24)
_scatter16 = _make_scatter(16)
_scatter8 = _make_scatter(8)
_dense1 = _make_dense(16, 24, 16)
_dense2 = _make_dense(24, 16, 32)
_dense3 = _make_dense(16, 8, 16)


def kernel(x, edge_index, edge_attr, batch, nn1_w1, nn1_b1, nn1_w2, nn1_b2,
           root1, bias1, nn2_w1, nn2_b1, nn2_w2, nn2_b2, root2, bias2,
           nn3_w1, nn3_b1, nn3_w2, nn3_b2, root3, bias3, lstm_wih, lstm_whh,
           lstm_bih, lstm_bhh, lin1_w, lin1_b, lin2_w, lin2_b, linf_w,
           linf_b):
  src = edge_index[0]
  dst = edge_index[1]
  zero24 = jnp.zeros((TS, 24), jnp.float32)
  zero16 = jnp.zeros((TS, 16), jnp.float32)
  zero8 = jnp.zeros((TS, 8), jnp.float32)
  row = lambda v: v.reshape(1, -1)
  dw1 = _dense_weights(16, 24, nn1_w2, nn1_b2)
  dw2 = _dense_weights(24, 16, nn2_w2, nn2_b2)
  dw3 = _dense_weights(16, 8, nn3_w2, nn3_b2)

  cacc = _scatter8(jnp.ones((E, 8), jnp.float32), dst, zero8)
  r1 = _root_call(24)(x, root1, row(bias1))
  xs1 = _gather16(x, src)
  msg1 = _dense1(edge_attr, xs1, nn1_w1, row(nn1_b1), *dw1)
  acc1 = _scatter24(msg1, dst, zero24)
  x1p, invc, r2 = _finish1(acc1, cacc, r1, root2, row(bias2))

  xs2 = _gather32(x1p, src)
  msg2 = _dense2(edge_attr, xs2, nn2_w1, row(nn2_b1), *dw2)
  acc2 = _scatter16(msg2, dst, zero16)
  x2, r3 = _finish2(acc2, invc, r2, root3, row(bias3))

  xs3 = _gather16(x2, src)
  msg3 = _dense3(edge_attr, xs3, nn3_w1, row(nn3_b1), *dw3)
  acc3 = _scatter8(msg3, dst, zero8)

  out = _s2s(acc3, invc, r3, batch.reshape(N, 1), lstm_wih, lstm_whh,
             row(lstm_bih + lstm_bhh), lin1_w, row(lin1_b), lin2_w,
             row(lin2_b), linf_w, row(linf_b))
  return out.reshape(B)


# speedup vs baseline: 3.3932x; 1.3255x over previous
import functools

import jax
import jax.numpy as jnp
from jax import lax
from jax.experimental import pallas as pl
from jax.experimental.pallas import tpu as pltpu
from jax.experimental.pallas import tpu_sc as plsc

N = 10000
E = 160000
B = 64

NC = 2
NS = 16
NW = NC * NS
CH = 128
GC = 10
GP = CH * GC
NG = E // GP
ACC_ROWS = 10016
TS = ACC_ROWS // NS
EB = 2000

_mesh = functools.partial(
    plsc.VectorSubcoreMesh, core_axis_name="c", subcore_axis_name="s",
    num_cores=NC, num_subcores=NS)


def _make_gather(width):
  npw = -(-NG // NW)

  @functools.partial(
      pl.kernel,
      out_type=jax.ShapeDtypeStruct((E, width), jnp.float32),
      mesh=_mesh(),
      scratch_types=[
          pltpu.VMEM((2, GC, CH), jnp.int32),
          pltpu.VMEM((2, GP, width), jnp.float32),
          pltpu.SemaphoreType.DMA,
          pltpu.SemaphoreType.DMA,
      ],
      compiler_params=pltpu.CompilerParams(use_tc_tiling_on_sc=False),
  )
  def gather(x_hbm, srci_hbm, xs_hbm, idx_v, rows_v, sem_i, sem_g):
    wid = lax.axis_index("s") * NC + lax.axis_index("c")
    groups = [wid + NW * t for t in range(npw)]

    def start_idx(t, g):
      pltpu.async_copy(srci_hbm.at[pl.ds(g * GC, GC)], idx_v.at[t % 2], sem_i)

    @pl.when(groups[0] < NG)
    def _():
      start_idx(0, groups[0])

    for t, g in enumerate(groups):
      b = t % 2

      @pl.when(g < NG)
      def _(t=t, g=g, b=b):
        pltpu.make_async_copy(
            srci_hbm.at[pl.ds(g * GC, GC)], idx_v.at[b], sem_i).wait()
        if t + 1 < npw:
          @pl.when(groups[t + 1] < NG)
          def _():
            start_idx(t + 1, groups[t + 1])
        descs = [
            pltpu.async_copy(x_hbm.at[idx_v.at[b, k]],
                             rows_v.at[b, pl.ds(k * CH, CH)], sem_g)
            for k in range(GC)
        ]
        for d in descs:
          d.wait()
        pltpu.sync_copy(rows_v.at[b], xs_hbm.at[pl.ds(g * GP, GP)])

  return gather


def _make_scatter(width):
  npt = -(-((NG + 1) // 2) // NS)

  @functools.partial(
      pl.kernel,
      out_type=jax.ShapeDtypeStruct((NC, ACC_ROWS, width), jnp.float32),
      mesh=_mesh(),
      scratch_types=[
          pltpu.VMEM((2, GC, CH), jnp.int32),
          pltpu.VMEM((2, GP, width), jnp.float32),
          pltpu.VMEM_SHARED((ACC_ROWS, width), jnp.float32),
          pltpu.SemaphoreType.DMA,
          pltpu.SemaphoreType.DMA,
      ],
      compiler_params=pltpu.CompilerParams(use_tc_tiling_on_sc=False),
  )
  def scatter(msg_hbm, dsti_hbm, zero_hbm, out_hbm, ibuf, mbuf, acc_sh,
              sem_l, sem_s):
    core = lax.axis_index("c")
    sid = lax.axis_index("s")

    pltpu.sync_copy(zero_hbm, acc_sh.at[pl.ds(sid * TS, TS)])

    gidx = [sid + NS * t for t in range(npt)]

    def start_loads(t):
      b = t % 2
      g = 2 * gidx[t] + core
      pltpu.async_copy(dsti_hbm.at[pl.ds(g * GC, GC)], ibuf.at[b], sem_l)
      pltpu.async_copy(msg_hbm.at[pl.ds(g * GP, GP)], mbuf.at[b], sem_l)

    @pl.when(2 * gidx[0] + core < NG)
    def _():
      start_loads(0)

    plsc.subcore_barrier()

    for t in range(npt):
      b = t % 2
      g = 2 * gidx[t] + core

      @pl.when(g < NG)
      def _(t=t, b=b, g=g):
        pltpu.make_async_copy(
            dsti_hbm.at[pl.ds(g * GC, GC)], ibuf.at[b], sem_l).wait()
        pltpu.make_async_copy(
            msg_hbm.at[pl.ds(g * GP, GP)], mbuf.at[b], sem_l).wait()
        if t + 1 < npt:
          @pl.when(2 * gidx[t + 1] + core < NG)
          def _():
            start_loads(t + 1)
        descs = [
            pltpu.async_copy(mbuf.at[b, pl.ds(k * CH, CH)],
                             acc_sh.at[ibuf.at[b, k]], sem_s, add=True)
            for k in range(GC)
        ]
        for d in descs:
          d.wait()

    plsc.subcore_barrier()
    pltpu.sync_copy(acc_sh.at[pl.ds(sid * TS, TS)],
                    out_hbm.at[core, pl.ds(sid * TS, TS)])

  return scatter


def _make_dense(cin, cout, xs_w):
  ku = cin * 32

  def body(ea_ref, xs_ref, w1_ref, b1_ref, rx_ref, rh_ref, w2r_ref, b2r_ref,
           o_ref):
    h = jnp.maximum(ea_ref[...] @ w1_ref[...] + b1_ref[...], 0.0)
    xc = xs_ref[...][:, :cin]
    u = (xc @ rx_ref[...]) * (h @ rh_ref[...])
    o_ref[...] = u @ w2r_ref[...] + xc @ b2r_ref[...]

  return pl.pallas_call(
      body,
      grid=(E // EB,),
      in_specs=[
          pl.BlockSpec((EB, 4), lambda i: (i, 0)),
          pl.BlockSpec((EB, xs_w), lambda i: (i, 0)),
          pl.BlockSpec((4, 32), lambda i: (0, 0)),
          pl.BlockSpec((1, 32), lambda i: (0, 0)),
          pl.BlockSpec((cin, ku), lambda i: (0, 0)),
          pl.BlockSpec((32, ku), lambda i: (0, 0)),
          pl.BlockSpec((ku, cout), lambda i: (0, 0)),
          pl.BlockSpec((cin, cout), lambda i: (0, 0)),
      ],
      out_specs=pl.BlockSpec((EB, cout), lambda i: (i, 0)),
      out_shape=jax.ShapeDtypeStruct((E, cout), jnp.float32),
  )


def _dense_weights(cin, cout, w2, b2):
  rx = jnp.kron(jnp.eye(cin, dtype=jnp.float32),
                jnp.ones((1, 32), jnp.float32))
  rh = jnp.kron(jnp.ones((1, cin), jnp.float32),
                jnp.eye(32, dtype=jnp.float32))
  w2r = w2.reshape(32, cin, cout).transpose(1, 0, 2).reshape(cin * 32, cout)
  b2r = b2.reshape(cin, cout)
  return rx, rh, w2r, b2r


def _root_body(x_ref, w_ref, b_ref, o_ref):
  o_ref[...] = x_ref[...] @ w_ref[...] + b_ref[...]


def _root_call(cout):
  return pl.pallas_call(
      _root_body, out_shape=jax.ShapeDtypeStruct((N, cout), jnp.float32))


def _finish1_body(acc_ref, cacc_ref, r1_ref, w_ref, b_ref, x1_ref, invc_ref,
                  r2_ref):
  a = acc_ref[0, :N, :] + acc_ref[1, :N, :]
  cnt = (cacc_ref[0, :N, 0:1] + cacc_ref[1, :N, 0:1])
  invc = 1.0 / jnp.maximum(cnt, 1.0)
  y = jnp.maximum(a * invc + r1_ref[...], 0.0)
  x1_ref[...] = jnp.concatenate([y, jnp.zeros((N, 8), jnp.float32)], axis=-1)
  invc_ref[...] = invc
  r2_ref[...] = y @ w_ref[...] + b_ref[...]


_finish1 = pl.pallas_call(
    _finish1_body,
    out_shape=(
        jax.ShapeDtypeStruct((N, 32), jnp.float32),
        jax.ShapeDtypeStruct((N, 1), jnp.float32),
        jax.ShapeDtypeStruct((N, 16), jnp.float32),
    ),
)


def _finish2_body(acc_ref, invc_ref, r2_ref, w_ref, b_ref, x2_ref, r3_ref):
  a = acc_ref[0, :N, :] + acc_ref[1, :N, :]
  y = jnp.maximum(a * invc_ref[...] + r2_ref[...], 0.0)
  x2_ref[...] = y
  r3_ref[...] = y @ w_ref[...] + b_ref[...]


_finish2 = pl.pallas_call(
    _finish2_body,
    out_shape=(
        jax.ShapeDtypeStruct((N, 16), jnp.float32),
        jax.ShapeDtypeStruct((N, 8), jnp.float32),
    ),
)


def _s2s_body(acc_ref, invc_ref, r3_ref, bt_ref, wih_ref, whh_ref, lb_ref,
              l1w_ref, l1b_ref, l2w_ref, l2b_ref, lfw_ref, lfb_ref, o_ref):
  a3 = acc_ref[0, :N, :] + acc_ref[1, :N, :]
  x3 = jnp.maximum(a3 * invc_ref[...] + r3_ref[...], 0.0)
  bt = bt_ref[...]
  cols = lax.broadcasted_iota(jnp.int32, (N, B), 1)
  oh = jnp.where(bt == cols, 1.0, 0.0)

  q_star = jnp.zeros((B, 16), jnp.float32)
  h = jnp.zeros((B, 8), jnp.float32)
  c = jnp.zeros((B, 8), jnp.float32)
  for _ in range(2):
    gates = q_star @ wih_ref[...] + h @ whh_ref[...] + lb_ref[...]
    ig = jax.nn.sigmoid(gates[:, 0:8])
    fg = jax.nn.sigmoid(gates[:, 8:16])
    gg = jnp.tanh(gates[:, 16:24])
    og = jax.nn.sigmoid(gates[:, 24:32])
    c = fg * c + ig * gg
    h = og * jnp.tanh(c)
    q = h
    qb = oh @ q
    e = jnp.sum(x3 * qb, axis=1, keepdims=True)
    masked = jnp.where(oh > 0.0, e, -jnp.inf)
    emax = jnp.max(masked, axis=0, keepdims=True)
    emax = jnp.where(jnp.isfinite(emax), emax, 0.0)
    emaxb = jnp.sum(oh * emax, axis=1, keepdims=True)
    aa = jnp.exp(e - emaxb)
    denom = jnp.sum(oh * aa, axis=0, keepdims=True)
    denomb = jnp.sum(oh * denom, axis=1, keepdims=True)
    aa = aa / (denomb + 1e-16)
    r = lax.dot_general(oh, aa * x3, (((0,), (0,)), ((), ())))
    q_star = jnp.concatenate([q, r], axis=-1)

  y = jnp.maximum(q_star @ l1w_ref[...] + l1b_ref[...], 0.0)
  y = jnp.maximum(y @ l2w_ref[...] + l2b_ref[...], 0.0)
  o_ref[...] = y @ lfw_ref[...] + lfb_ref[...]


_s2s = pl.pallas_call(
    _s2s_body, out_shape=jax.ShapeDtypeStruct((B, 1), jnp.float32))


_gather16 = _make_gather(16)
_gather32 = _make_gather(32)
_scatter24 = _make_scatter(24)
_scatter16 = _make_scatter(16)
_scatter8 = _make_scatter(8)
_dense1 = _make_dense(16, 24, 16)
_dense2 = _make_dense(24, 16, 32)
_dense3 = _make_dense(16, 8, 16)


def kernel(x, edge_index, edge_attr, batch, nn1_w1, nn1_b1, nn1_w2, nn1_b2,
           root1, bias1, nn2_w1, nn2_b1, nn2_w2, nn2_b2, root2, bias2,
           nn3_w1, nn3_b1, nn3_w2, nn3_b2, root3, bias3, lstm_wih, lstm_whh,
           lstm_bih, lstm_bhh, lin1_w, lin1_b, lin2_w, lin2_b, linf_w,
           linf_b):
  src = edge_index[0].reshape(E // CH, CH)
  dst = edge_index[1].reshape(E // CH, CH)
  zero24 = jnp.zeros((TS, 24), jnp.float32)
  zero16 = jnp.zeros((TS, 16), jnp.float32)
  zero8 = jnp.zeros((TS, 8), jnp.float32)
  row = lambda v: v.reshape(1, -1)
  dw1 = _dense_weights(16, 24, nn1_w2, nn1_b2)
  dw2 = _dense_weights(24, 16, nn2_w2, nn2_b2)
  dw3 = _dense_weights(16, 8, nn3_w2, nn3_b2)

  cacc = _scatter8(jnp.ones((E, 8), jnp.float32), dst, zero8)
  r1 = _root_call(24)(x, root1, row(bias1))
  xs1 = _gather16(x, src)
  msg1 = _dense1(edge_attr, xs1, nn1_w1, row(nn1_b1), *dw1)
  acc1 = _scatter24(msg1, dst, zero24)
  x1p, invc, r2 = _finish1(acc1, cacc, r1, root2, row(bias2))

  xs2 = _gather32(x1p, src)
  msg2 = _dense2(edge_attr, xs2, nn2_w1, row(nn2_b1), *dw2)
  acc2 = _scatter16(msg2, dst, zero16)
  x2, r3 = _finish2(acc2, invc, r2, root3, row(bias3))

  xs3 = _gather16(x2, src)
  msg3 = _dense3(edge_attr, xs3, nn3_w1, row(nn3_b1), *dw3)
  acc3 = _scatter8(msg3, dst, zero8)

  out = _s2s(acc3, invc, r3, batch.reshape(N, 1), lstm_wih, lstm_whh,
             row(lstm_bih + lstm_bhh), lin1_w, row(lin1_b), lin2_w,
             row(lin2_b), linf_w, row(linf_b))
  return out.reshape(B)
